# baseline (speedup 1.0000x reference)
import functools

import jax
import jax.numpy as jnp
from jax.experimental import pallas as pl
from jax.experimental.pallas import tpu as pltpu


def _nn_body(e_ref, c_ref, out_ref, ca_ref, *, tp: int, n_chunks: int):
    i = pl.program_id(0)
    d = e_ref.shape[1]

    @pl.when(i == 0)
    def _init():
        c = c_ref[...]
        ca_ref[:, :d] = c.astype(jnp.bfloat16)
        cn = jnp.sum(c * c, axis=1, keepdims=True)
        ca_ref[:, d:] = (-cn).astype(jnp.bfloat16)

    e = e_ref[...]
    en = jnp.sum(e * e, axis=1, keepdims=True)
    e_aug = jnp.concatenate(
        [e + e, jnp.ones((e.shape[0], 1), jnp.float32)],
        axis=1).astype(jnp.bfloat16)

    def step(k, bw):
        ca = ca_ref[pl.ds(k * tp, tp), :]
        s = jax.lax.dot_general(
            e_aug, ca, (((1,), (1,)), ((), ())),
            preferred_element_type=jnp.float32
            ).astype(jnp.bfloat16)
        m = jnp.maximum(s[:, 0:128], s[:, 128:256])
        for j in range(2, tp // 128):
            m = jnp.maximum(m, s[:, j * 128:(j + 1) * 128])
        return jnp.maximum(bw, m)

    bw = jax.lax.fori_loop(
        0, n_chunks, step,
        jnp.full((e.shape[0], 128), -jnp.inf, dtype=jnp.bfloat16),
        unroll=True)
    best = jnp.max(bw, axis=1, keepdims=True
                   ).astype(jnp.float32)
    out_ref[...] = jnp.sqrt(jnp.maximum(en - best, 0.0))


def kernel(embeds, centroids, r):
    b, n, d = embeds.shape
    p = centroids.shape[0]
    h = int(round(n ** 0.5))
    nq = b * n

    tq = 3136
    tp = 2048

    eq = embeds.reshape(nq, d)
    out = pl.pallas_call(
        functools.partial(_nn_body, tp=tp, n_chunks=p // tp),
        grid=(nq // tq,),
        in_specs=[
            pl.BlockSpec((tq, d), lambda i: (i, 0)),
            pl.BlockSpec((p, d), lambda i: (0, 0)),
        ],
        out_specs=pl.BlockSpec((tq, 1), lambda i: (i, 0)),
        out_shape=jax.ShapeDtypeStruct((nq, 1), jnp.float32),
        scratch_shapes=[
            pltpu.VMEM((p, d + 1), jnp.bfloat16),
        ],
    )(eq, centroids)

    score = out.reshape(b, h, h)[:, None, :, :]
    return (jnp.float32(0.0), score)


# speedup vs baseline: 1.0036x; 1.0036x over previous
import functools

import jax
import jax.numpy as jnp
from jax.experimental import pallas as pl
from jax.experimental.pallas import tpu as pltpu


def _nn_body(e_ref, c_ref, out_ref, ca_ref, *, tp: int, n_chunks: int):
    i = pl.program_id(0)
    d = e_ref.shape[1]

    @pl.when(i == 0)
    def _init():
        c = c_ref[...]
        ca_ref[:, :d] = c.astype(jnp.bfloat16)
        cn = jnp.sum(c * c, axis=1, keepdims=True)
        ca_ref[:, d:] = (-cn).astype(jnp.bfloat16)

    e = e_ref[...]
    en = jnp.sum(e * e, axis=1, keepdims=True)
    e_aug = jnp.concatenate(
        [e + e, jnp.ones((e.shape[0], 1), jnp.float32)],
        axis=1).astype(jnp.bfloat16)

    def step(k, bw):
        ca = ca_ref[pl.ds(k * tp, tp), :]
        s = jax.lax.dot_general(
            e_aug, ca, (((1,), (1,)), ((), ())),
            preferred_element_type=jnp.float32)
        m = jnp.maximum(s[:, 0:128], s[:, 128:256])
        for j in range(2, tp // 128):
            m = jnp.maximum(m, s[:, j * 128:(j + 1) * 128])
        return jnp.maximum(bw, m)

    bw = jax.lax.fori_loop(
        0, n_chunks, step,
        jnp.full((e.shape[0], 128), -jnp.inf, dtype=jnp.float32),
        unroll=True)
    best = jnp.max(bw, axis=1, keepdims=True)
    out_ref[...] = jnp.sqrt(jnp.maximum(en - best, 0.0))


def kernel(embeds, centroids, r):
    b, n, d = embeds.shape
    p = centroids.shape[0]
    h = int(round(n ** 0.5))
    nq = b * n

    tq = 3136
    tp = 1024

    eq = embeds.reshape(nq, d)
    out = pl.pallas_call(
        functools.partial(_nn_body, tp=tp, n_chunks=p // tp),
        grid=(nq // tq,),
        in_specs=[
            pl.BlockSpec((tq, d), lambda i: (i, 0)),
            pl.BlockSpec((p, d), lambda i: (0, 0)),
        ],
        out_specs=pl.BlockSpec((tq, 1), lambda i: (i, 0)),
        out_shape=jax.ShapeDtypeStruct((nq, 1), jnp.float32),
        scratch_shapes=[
            pltpu.VMEM((p, d + 1), jnp.bfloat16),
        ],
    )(eq, centroids)

    score = out.reshape(b, h, h)[:, None, :, :]
    return (jnp.float32(0.0), score)
